# Initial kernel scaffold; baseline (speedup 1.0000x reference)
import jax
import jax.numpy as jnp
from jax.experimental import pallas as pl


def kernel(tokens, W_E):
    raise NotImplementedError("write your pallas kernel here")


# speedup vs baseline: 1.7182x; 1.7182x over previous
import functools

import jax
from jax import lax
import jax.numpy as jnp
from jax.experimental import pallas as pl
from jax.experimental.pallas import tpu as pltpu
from jax.experimental.pallas import tpu_sc as plsc

D_MODEL = 768
CHUNK = 64


def _embed_sc(tokens_flat, W_E, B):
    info = plsc.get_sparse_core_info()
    nw = info.num_cores * info.num_subcores
    b_per_w = B // nw
    nchunks = b_per_w // CHUNK
    mesh = plsc.VectorSubcoreMesh(core_axis_name="core",
                                  subcore_axis_name="subcore")

    @functools.partial(
        pl.kernel,
        out_type=jax.ShapeDtypeStruct((B, D_MODEL), W_E.dtype),
        mesh=mesh,
        scratch_types=[
            pltpu.VMEM((b_per_w,), jnp.int32),
            pltpu.VMEM((CHUNK, D_MODEL), jnp.float32),
            pltpu.VMEM((CHUNK, D_MODEL), jnp.float32),
            pltpu.SemaphoreType.DMA,
            pltpu.SemaphoreType.DMA,
            pltpu.SemaphoreType.DMA,
            pltpu.SemaphoreType.DMA,
        ],
    )
    def k(table_hbm, idx_hbm, out_hbm, idx_v, rows0, rows1,
          gsem0, gsem1, ssem0, ssem1):
        wid = lax.axis_index("subcore") * info.num_cores + lax.axis_index("core")
        base = wid * b_per_w
        pltpu.sync_copy(idx_hbm.at[pl.ds(base, b_per_w)], idx_v)

        bufs = ((rows0, gsem0, ssem0), (rows1, gsem1, ssem1))

        def gather(c, buf, gsem):
            return pltpu.make_async_copy(
                table_hbm.at[idx_v.at[pl.ds(c * CHUNK, CHUNK)]], buf, gsem)

        def store(c, buf, ssem):
            return pltpu.make_async_copy(
                buf, out_hbm.at[pl.ds(base + c * CHUNK, CHUNK)], ssem)

        for s, (buf, gsem, _) in enumerate(bufs):
            gather(s, buf, gsem).start()

        @pl.loop(0, nchunks, step=2)
        def _(c):
            for s, (buf, gsem, ssem) in enumerate(bufs):
                cc = c + s
                gather(cc, buf, gsem).wait()
                store(cc, buf, ssem).start()
                store(cc, buf, ssem).wait()

                @pl.when(cc + 2 < nchunks)
                def _():
                    gather(cc + 2, buf, gsem).start()

    return k(W_E, tokens_flat)


def kernel(tokens, W_E):
    n_batch, seq = tokens.shape
    B = n_batch * seq
    out = _embed_sc(tokens.reshape(B), W_E, B)
    return out.reshape(n_batch, seq, D_MODEL)
